# Initial kernel scaffold; baseline (speedup 1.0000x reference)
import jax
import jax.numpy as jnp
from jax.experimental import pallas as pl


def kernel(x, edge_index1, pseudo1, cluster1, edge_index2, pseudo2, cluster2, edge_index3, pseudo3, cluster3, edge_index4, pseudo4, params):
    raise NotImplementedError("write your pallas kernel here")


# speedup vs baseline: 2.1427x; 2.1427x over previous
import jax
import jax.numpy as jnp
import numpy as np
from jax import lax
from jax.experimental import pallas as pl
from jax.experimental.pallas import tpu as pltpu
from jax.experimental.pallas import tpu_sc as plsc

K = 5
DIM = 3
KP = K ** DIM
NC, NS = 2, 16
NW = NC * NS
EB = 128
LW = 128

_BITS = [[(c >> d) & 1 for d in range(DIM)] for c in range(2 ** DIM)]


def _rup(v, m):
  return ((v + m - 1) // m) * m


def _m8(v):
  return pl.multiple_of(v, 8)


def _sc_mesh():
  return plsc.VectorSubcoreMesh(
      core_axis_name="c", subcore_axis_name="s", num_cores=NC,
      num_subcores=NS)


_SC_PARAMS = pltpu.CompilerParams(needs_layout_passes=False)


def _tc_prep(src2, p02, p12, p22):
  r = src2.shape[0]

  def body(s_ref, p0_ref, p1_ref, p2_ref, bc_ref, gi_ref):
    s = s_ref[...]
    bots, fracs = [], []
    for pr in (p0_ref, p1_ref, p2_ref):
      v = pr[...] * np.float32(K - 1)
      b = jnp.clip(v.astype(jnp.int32), 0, K - 2)
      bots.append(b)
      fracs.append(v - b.astype(jnp.float32))
    base = s * KP
    for t, bits in enumerate(_BITS):
      w = jnp.ones_like(fracs[0])
      wi = jnp.zeros_like(s)
      for d, m in enumerate((1, K, K * K)):
        w = w * (fracs[d] if bits[d] else 1.0 - fracs[d])
        wi = wi + (bots[d] + bits[d]) * m
      bc_ref[t] = w
      gi_ref[t] = base + wi

  return pl.pallas_call(
      body,
      grid=(r // 8,),
      in_specs=[pl.BlockSpec((8, 128), lambda i: (i, 0))] * 4,
      out_specs=[
          pl.BlockSpec((8, 8, 128), lambda i: (0, i, 0)),
          pl.BlockSpec((8, 8, 128), lambda i: (0, i, 0)),
      ],
      out_shape=[
          jax.ShapeDtypeStruct((8, r, 128), jnp.float32),
          jax.ShapeDtypeStruct((8, r, 128), jnp.int32),
      ],
  )(src2, p02, p12, p22)


def _sc_conv(y, gidx, bc, dst, npad, ng):
  epad = dst.shape[0]
  ew = epad // NW
  nchunks = ew // EB
  rchunk = ew // 128
  rz = npad // NS
  aw = LW

  def body(y_h, gi_h, bc_h, dst_h, part_h,
           idxb, bcb, dstb, rows, msg, acc, sem):
    cid = lax.axis_index("c")
    sid = lax.axis_index("s")
    wid = sid * NC + cid

    def zfill(b, _):
      for g in range(8):
        msg[b, pl.ds(g * 16, 16)] = jnp.zeros((16,), jnp.float32)
      return 0
    lax.fori_loop(0, EB, zfill, 0)

    def zacc(j, _):
      pltpu.sync_copy(msg.at[pl.ds(0, 16)],
                      acc.at[pl.ds(_m8(sid * rz + j * 16), 16)])
      return 0
    lax.fori_loop(0, rz // 16, zacc, 0)
    plsc.subcore_barrier()

    def chunk(ci, _):
      e0 = wid * ew + ci * EB
      r0 = wid * rchunk + ci
      pltpu.sync_copy(gi_h.at[r0], idxb)
      pltpu.sync_copy(bc_h.at[pl.ds(_m8(e0 * 8), EB * 8)],
                      bcb.at[pl.ds(0, EB * 8)])
      pltpu.sync_copy(dst_h.at[pl.ds(_m8(e0), EB)], dstb)
      for rnd in range(8):
        pltpu.async_copy(y_h.at[idxb.at[rnd]], rows, sem).wait()

        def edge(b, _):
          bv = bcb[pl.ds(b * 8, 16)]
          for g in range(ng):
            a = bv[rnd] * rows[b, pl.ds(g * 16, 16)]
            if rnd:
              a = a + msg[b, pl.ds(g * 16, 16)]
            msg[b, pl.ds(g * 16, 16)] = a
          return 0

        lax.fori_loop(0, EB, edge, 0)
      pltpu.sync_copy(msg, acc.at[dstb], add=True)
      return 0

    lax.fori_loop(0, nchunks, chunk, 0)
    plsc.subcore_barrier()
    pltpu.sync_copy(acc.at[pl.ds(_m8(sid * rz), rz)],
                    part_h.at[cid, pl.ds(_m8(sid * rz), rz)])

  return pl.kernel(
      body,
      out_type=jax.ShapeDtypeStruct((NC, npad, aw), jnp.float32),
      mesh=_sc_mesh(),
      compiler_params=_SC_PARAMS,
      scratch_types=[
          pltpu.VMEM((8, 128), jnp.int32),
          pltpu.VMEM((EB * 8 + 16,), jnp.float32),
          pltpu.VMEM((EB,), jnp.int32),
          pltpu.VMEM((EB, LW), jnp.float32),
          pltpu.VMEM((EB, aw), jnp.float32),
          pltpu.VMEM_SHARED((npad, aw), jnp.float32),
          pltpu.SemaphoreType.DMA,
      ],
  )(y, gidx, bc, dst)


def _sc_gather(table, idx):
  n = idx.shape[0]
  d = table.shape[1]
  bpw = n // NW
  bu = bpw if bpw <= 128 else 64
  nbu = bpw // bu

  def body(tab_h, idx_h, out_h, idxb, rowsb, sem):
    cid = lax.axis_index("c")
    sid = lax.axis_index("s")
    wid = sid * NC + cid
    base = wid * bpw
    pltpu.sync_copy(idx_h.at[wid], idxb)
    cps = []
    for j in range(nbu):
      cps.append(pltpu.async_copy(
          tab_h.at[idxb.at[j]], rowsb.at[pl.ds(j * bu, bu)], sem))
    for cp in cps:
      cp.wait()
    pltpu.sync_copy(rowsb, out_h.at[pl.ds(_m8(base), bpw)])

  idx3 = idx.reshape(NW, nbu, bu)
  return pl.kernel(
      body,
      out_type=jax.ShapeDtypeStruct((n, d), jnp.float32),
      mesh=_sc_mesh(),
      compiler_params=_SC_PARAMS,
      scratch_types=[
          pltpu.VMEM((nbu, bu), jnp.int32),
          pltpu.VMEM((bpw, d), jnp.float32),
          pltpu.SemaphoreType.DMA,
      ],
  )(table, idx3)


def _tc_matmul(xp, w, bm, bn):
  npad = xp.shape[0]
  kout = w.shape[1]

  def body(x_ref, w_ref, o_ref):
    o_ref[...] = jnp.dot(x_ref[...], w_ref[...],
                         preferred_element_type=jnp.float32)

  return pl.pallas_call(
      body,
      grid=(npad // bm, kout // bn),
      in_specs=[
          pl.BlockSpec((bm, 128), lambda i, j: (i, 0)),
          pl.BlockSpec((128, bn), lambda i, j: (0, j)),
      ],
      out_specs=pl.BlockSpec((bm, bn), lambda i, j: (i, j)),
      out_shape=jax.ShapeDtypeStruct((npad, kout), jnp.float32),
  )(xp, w)


def _tc_epilogue(p0, p1, xp, wr, out_c, bm):
  npad = xp.shape[0]
  aw = p0.shape[1]

  def body(p0_ref, p1_ref, x_ref, w_ref, o_ref):
    s = p0_ref[...] + p1_ref[...]
    lanes_a = lax.broadcasted_iota(jnp.int32, (bm, aw), 1)
    deg = jnp.sum(jnp.where(lanes_a == out_c, s, 0.0), axis=1,
                  keepdims=True)
    scale = 1.0 / jnp.maximum(deg, 1.0)
    agg = s * scale
    if aw < LW:
      agg = jnp.concatenate(
          [agg, jnp.zeros((bm, LW - aw), jnp.float32)], axis=1)
    root = jnp.dot(x_ref[...], w_ref[...],
                   preferred_element_type=jnp.float32)
    h = agg + root
    h = jnp.where(h > 0, h, jnp.exp(h) - 1.0)
    lanes = lax.broadcasted_iota(jnp.int32, (bm, LW), 1)
    o_ref[...] = jnp.where(lanes >= out_c, 1.0, h)

  return pl.pallas_call(
      body,
      grid=(npad // bm,),
      in_specs=[
          pl.BlockSpec((bm, aw), lambda i: (i, 0)),
          pl.BlockSpec((bm, aw), lambda i: (i, 0)),
          pl.BlockSpec((bm, LW), lambda i: (i, 0)),
          pl.BlockSpec((LW, LW), lambda i: (0, 0)),
      ],
      out_specs=pl.BlockSpec((bm, LW), lambda i: (i, 0)),
      out_shape=jax.ShapeDtypeStruct((npad, LW), jnp.float32),
  )(p0, p1, xp, wr)


def _tc_pool(xp, cluster, n_in, n_outpad, c_out):
  n_inpad = xp.shape[0]

  def body(cl_ref, x_ref, o_ref):
    o_ref[...] = jnp.full((n_outpad, LW), -jnp.inf, jnp.float32)

    def step(i, _):
      j = cl_ref[i]
      row = jnp.maximum(o_ref[pl.ds(j, 1), :], x_ref[pl.ds(i, 1), :])
      o_ref[pl.ds(j, 1), :] = row
      return 0

    lax.fori_loop(0, n_in, step, 0)
    v = o_ref[...]
    v = jnp.where(jnp.isfinite(v), v, 0.0)
    lanes = lax.broadcasted_iota(jnp.int32, (n_outpad, LW), 1)
    o_ref[...] = jnp.where(lanes >= c_out, 1.0, v)

  return pl.pallas_call(
      body,
      in_specs=[
          pl.BlockSpec(memory_space=pltpu.SMEM),
          pl.BlockSpec((n_inpad, LW), lambda: (0, 0)),
      ],
      out_specs=pl.BlockSpec((n_outpad, LW), lambda: (0, 0)),
      out_shape=jax.ShapeDtypeStruct((n_outpad, LW), jnp.float32),
  )(cluster, xp)


def _tc_skipadd(up, xp, w, bm):
  npad = xp.shape[0]

  def body(u_ref, x_ref, w_ref, o_ref):
    o_ref[...] = u_ref[...] + jnp.dot(x_ref[...], w_ref[...],
                                      preferred_element_type=jnp.float32)

  return pl.pallas_call(
      body,
      grid=(npad // bm,),
      in_specs=[
          pl.BlockSpec((bm, LW), lambda i: (i, 0)),
          pl.BlockSpec((bm, LW), lambda i: (i, 0)),
          pl.BlockSpec((LW, LW), lambda i: (0, 0)),
      ],
      out_specs=pl.BlockSpec((bm, LW), lambda i: (i, 0)),
      out_shape=jax.ShapeDtypeStruct((npad, LW), jnp.float32),
  )(up, xp, w)


def _tc_fc1(xp, w, c_out, bm):
  npad = xp.shape[0]

  def body(x_ref, w_ref, o_ref):
    t = jnp.dot(x_ref[...], w_ref[...], preferred_element_type=jnp.float32)
    t = jnp.where(t > 0, t, jnp.exp(t) - 1.0)
    lanes = lax.broadcasted_iota(jnp.int32, (bm, LW), 1)
    o_ref[...] = jnp.where(lanes >= c_out, 1.0, t)

  return pl.pallas_call(
      body,
      grid=(npad // bm,),
      in_specs=[
          pl.BlockSpec((bm, LW), lambda i: (i, 0)),
          pl.BlockSpec((LW, LW), lambda i: (0, 0)),
      ],
      out_specs=pl.BlockSpec((bm, LW), lambda i: (i, 0)),
      out_shape=jax.ShapeDtypeStruct((npad, LW), jnp.float32),
  )(xp, w)


def _tc_fc2ls(xp, w, ncls, bm):
  npad = xp.shape[0]

  def body(x_ref, w_ref, o_ref):
    t = jnp.dot(x_ref[...], w_ref[...], preferred_element_type=jnp.float32)
    lanes = lax.broadcasted_iota(jnp.int32, (bm, LW), 1)
    masked = jnp.where(lanes < ncls, t, -jnp.inf)
    m = jnp.max(masked, axis=1, keepdims=True)
    e = jnp.exp(masked - m)
    ssum = jnp.sum(e, axis=1, keepdims=True)
    o_ref[...] = t - m - jnp.log(ssum)

  return pl.pallas_call(
      body,
      grid=(npad // bm,),
      in_specs=[
          pl.BlockSpec((bm, LW), lambda i: (i, 0)),
          pl.BlockSpec((LW, LW), lambda i: (0, 0)),
      ],
      out_specs=pl.BlockSpec((bm, LW), lambda i: (i, 0)),
      out_shape=jax.ShapeDtypeStruct((npad, LW), jnp.float32),
  )(xp, w)


def _pack_conv(wp):
  w, wr, b = wp
  kp, in_c, out_c = w.shape
  wf = jnp.zeros((LW, KP, LW), jnp.float32)
  wf = wf.at[:in_c, :, :out_c].set(jnp.transpose(w, (1, 0, 2)))
  wf = wf.at[in_c, :, out_c].set(1.0)
  wf = wf.reshape(LW, KP * LW)
  wr128 = jnp.zeros((LW, LW), jnp.float32)
  wr128 = wr128.at[:in_c, :out_c].set(wr)
  wr128 = wr128.at[in_c, :out_c].set(b)
  return wf, wr128, out_c


def _pack_lin(lp, in_c):
  w, b = lp
  out_c = w.shape[1]
  wa = jnp.zeros((LW, LW), jnp.float32)
  wa = wa.at[:in_c, :out_c].set(w)
  wa = wa.at[in_c, :out_c].set(b)
  return wa, out_c


class _Level:
  def __init__(self, n, edge_index, pseudo):
    e = edge_index.shape[1]
    self.n = n
    self.npad = _rup(n + 1, 256)
    self.epad = _rup(e, NW * EB)
    pe = self.epad - e
    src = edge_index[0].astype(jnp.int32)
    dst = edge_index[1].astype(jnp.int32)
    self.src = jnp.pad(src, (0, pe))
    self.dst = jnp.pad(dst, (0, pe), constant_values=n)
    ps = pseudo.astype(jnp.float32)
    self.p = [jnp.pad(ps[:, d], (0, pe)) for d in range(DIM)]
    self.gidx = None
    self.bc = None

  def prep(self):
    r = self.epad // 128
    bc_tm, gi_tm = _tc_prep(self.src.reshape(r, 128),
                            self.p[0].reshape(r, 128),
                            self.p[1].reshape(r, 128),
                            self.p[2].reshape(r, 128))
    self.gidx = jnp.transpose(gi_tm, (1, 0, 2))
    self.bc = jnp.transpose(bc_tm, (1, 2, 0)).reshape(-1)


def _conv(lv, xp, wp, bm=256):
  wf, wr, out_c = _pack_conv(wp)
  y = _tc_matmul(xp, wf, bm, 3200).reshape(lv.npad * KP, LW)
  part = _sc_conv(y, lv.gidx, lv.bc, lv.dst, lv.npad, out_c // 16 + 1)
  return _tc_epilogue(part[0], part[1], xp, wr, out_c, bm)


def kernel(x, edge_index1, pseudo1, cluster1, edge_index2, pseudo2,
           cluster2, edge_index3, pseudo3, cluster3, edge_index4, pseudo4,
           params):
  n1, n2, n3, n4 = 10000, 2500, 625, 160
  lv1 = _Level(n1, edge_index1, pseudo1)
  lv2 = _Level(n2, edge_index2, pseudo2)
  lv3 = _Level(n3, edge_index3, pseudo3)
  lv4 = _Level(n4, edge_index4, pseudo4)
  for lv in (lv1, lv2, lv3, lv4):
    lv.prep()

  c1 = jnp.pad(cluster1.astype(jnp.int32), (0, lv1.npad - n1))
  c2 = jnp.pad(cluster2.astype(jnp.int32), (0, lv2.npad - n2))
  c3 = jnp.pad(cluster3.astype(jnp.int32), (0, lv3.npad - n3))

  x0 = jnp.pad(x.astype(jnp.float32), ((0, lv1.npad - n1), (0, 0)))
  xp1 = jnp.concatenate(
      [x0, jnp.ones((lv1.npad, LW - 1), jnp.float32)], axis=1)

  p = params
  h1 = _conv(lv1, xp1, p["conv1"])
  h1 = _conv(lv1, h1, p["conv12"])
  h2 = _tc_pool(h1, c1, n1, lv2.npad, 64)
  h2 = _conv(lv2, h2, p["conv2"])
  h2 = _conv(lv2, h2, p["conv22"])
  h3 = _tc_pool(h2, c2, n2, lv3.npad, 64)
  h3 = _conv(lv3, h3, p["conv3"])
  h3 = _conv(lv3, h3, p["conv32"])
  h4 = _tc_pool(h3, c3, n3, lv4.npad, 64)
  h4 = _conv(lv4, h4, p["conv4"])
  h4 = _conv(lv4, h4, p["conv42"])
  h4 = _conv(lv4, h4, p["conv42"])

  wfc1, cfc1 = _pack_lin(p["fc1"], 64)
  h4 = _tc_fc1(h4, wfc1, cfc1, 256)

  ws3, _ = _pack_lin(p["skip3"], 64)
  up3 = _sc_gather(h4, c3)
  h3 = _tc_skipadd(up3, h3, ws3, 256)
  h3 = _conv(lv3, h3, p["conv5"])

  ws2, _ = _pack_lin(p["skip2"], 64)
  up2 = _sc_gather(h3, c2)
  h2 = _tc_skipadd(up2, h2, ws2, 256)
  h2 = _conv(lv2, h2, p["conv6"])

  ws1, _ = _pack_lin(p["skip1"], 64)
  up1 = _sc_gather(h2, c1)
  h1 = _tc_skipadd(up1, h1, ws1, 256)
  h1 = _conv(lv1, h1, p["conv7"])

  wfc2, _ = _pack_lin(p["fc2"], 32)
  out = _tc_fc2ls(h1, wfc2, 3, 256)
  return out[:n1, :3]
